# baseline (speedup 1.0000x reference)
import functools

import jax
import jax.numpy as jnp
from jax import lax
from jax.experimental import pallas as pl
from jax.experimental.pallas import tpu as pltpu
from jax.experimental.pallas import tpu_sc as plsc

_B = 16384
_D = 64
_K = 5
_NC = 2
_NS = 16
_NW = _NC * _NS
_BPW = _B // _NW
_CHUNK = 64
_NCHUNK = _BPW // _CHUNK
_LANES = 16
_LG = _CHUNK // _LANES
_PD = 2 * _D


def _sc_scores(targets, contexts, neg_t, w_hidden2, w_output2):
    mesh = plsc.VectorSubcoreMesh(
        core_axis_name="c", subcore_axis_name="s",
        num_cores=_NC, num_subcores=_NS)

    @functools.partial(
        pl.kernel,
        out_type=(
            jax.ShapeDtypeStruct((_B,), jnp.float32),
            jax.ShapeDtypeStruct((_B,), jnp.float32),
        ),
        mesh=mesh,
        scratch_types=[
            pltpu.VMEM((_CHUNK,), jnp.int32),
            pltpu.VMEM((_CHUNK,), jnp.int32),
            pltpu.VMEM((_K, _CHUNK), jnp.int32),
            pltpu.VMEM((_CHUNK,), jnp.int32),
            pltpu.VMEM((_CHUNK,), jnp.int32),
            pltpu.VMEM((_CHUNK,), jnp.int32),
            pltpu.VMEM((_CHUNK,), jnp.int32),
            pltpu.VMEM((_K, _CHUNK), jnp.int32),
            pltpu.VMEM((_K, _CHUNK), jnp.int32),
            pltpu.VMEM((_CHUNK, _PD), jnp.float32),
            pltpu.VMEM((_CHUNK, _PD), jnp.float32),
            pltpu.VMEM((_K, _CHUNK, _PD), jnp.float32),
            pltpu.VMEM((_CHUNK,), jnp.float32),
            pltpu.VMEM((_CHUNK,), jnp.float32),
            pltpu.SemaphoreType.DMA,
        ],
        compiler_params=pltpu.CompilerParams(needs_layout_passes=False),
    )
    def sc_kernel(tgt_hbm, ctx_hbm, negt_hbm, wh_hbm, wo_hbm, pos_out, neg_out,
                  tgt_v, ctx_v, negi_v, tgtl_v, tgto_v, ctxl_v, ctxo_v,
                  negl_v, nego_v, h_v, o_v, n_v, ps_v, ns_v, sem):
        wid = lax.axis_index("s") * _NC + lax.axis_index("c")

        def chunk_body(c, carry):
            base = wid * _BPW + c * _CHUNK
            pltpu.sync_copy(tgt_hbm.at[pl.ds(base, _CHUNK)], tgt_v)
            pltpu.sync_copy(ctx_hbm.at[pl.ds(base, _CHUNK)], ctx_v)
            for kk in range(_K):
                pltpu.sync_copy(negt_hbm.at[kk, pl.ds(base, _CHUNK)], negi_v.at[kk])
            for g in range(_LG):
                sl = pl.ds(g * _LANES, _LANES)
                t = tgt_v[sl]
                tgtl_v[sl] = lax.shift_right_logical(t, 1)
                tgto_v[sl] = lax.shift_left(jnp.bitwise_and(t, 1), 6)
                u = ctx_v[sl]
                ctxl_v[sl] = lax.shift_right_logical(u, 1)
                ctxo_v[sl] = lax.shift_left(jnp.bitwise_and(u, 1), 6)
                for kk in range(_K):
                    w = negi_v[kk, sl]
                    negl_v[kk, sl] = lax.shift_right_logical(w, 1)
                    nego_v[kk, sl] = lax.shift_left(jnp.bitwise_and(w, 1), 6)
            copies = [
                pltpu.make_async_copy(wh_hbm.at[tgtl_v], h_v, sem),
                pltpu.make_async_copy(wo_hbm.at[ctxl_v], o_v, sem),
            ]
            for kk in range(_K):
                copies.append(
                    pltpu.make_async_copy(wo_hbm.at[negl_v.at[kk]], n_v.at[kk], sem))
            for cp in copies:
                cp.start()
            for cp in copies:
                cp.wait()

            for g in range(_LG):
                sl = pl.ds(g * _LANES, _LANES)
                rows = lax.iota(jnp.int32, _LANES) + g * _LANES
                oh = tgto_v[sl]
                oo = ctxo_v[sl]
                on = [nego_v[kk, sl] for kk in range(_K)]
                kfull = [jnp.full((_LANES,), kk, jnp.int32) for kk in range(_K)]

                def d_body(d, accs):
                    acc_p, acc_n = accs
                    h = plsc.load_gather(h_v, [rows, oh + d])
                    o = plsc.load_gather(o_v, [rows, oo + d])
                    nsum = plsc.load_gather(n_v, [kfull[0], rows, on[0] + d])
                    for kk in range(1, _K):
                        nsum = nsum + plsc.load_gather(
                            n_v, [kfull[kk], rows, on[kk] + d])
                    return acc_p + h * o, acc_n + h * nsum

                acc_p, acc_n = lax.fori_loop(
                    0, _D, d_body,
                    (jnp.zeros((_LANES,), jnp.float32),
                     jnp.zeros((_LANES,), jnp.float32)))
                ps_v[sl] = acc_p
                ns_v[sl] = acc_n

            pltpu.sync_copy(ps_v, pos_out.at[pl.ds(base, _CHUNK)])
            pltpu.sync_copy(ns_v, neg_out.at[pl.ds(base, _CHUNK)])
            return carry

        lax.fori_loop(0, _NCHUNK, chunk_body, 0)

    return sc_kernel(targets, contexts, neg_t, w_hidden2, w_output2)


def _log_sigmoid(x):
    return jnp.minimum(x, 0.0) - jnp.log1p(jnp.exp(-jnp.abs(x)))


def _tc_loss(pos_score, neg_score, *, interpret=False):
    def body(p_ref, n_ref, o_ref):
        total = (jnp.sum(_log_sigmoid(p_ref[...]))
                 + jnp.sum(_log_sigmoid(-n_ref[...])))
        o_ref[0, 0] = -total * (1.0 / _B)

    return pl.pallas_call(
        body,
        out_shape=jax.ShapeDtypeStruct((1, 1), jnp.float32),
        out_specs=pl.BlockSpec(memory_space=pltpu.SMEM),
        interpret=interpret,
    )(pos_score, neg_score)


def kernel(targets, contexts, neg_samples, W_hidden, W_output):
    tgt = targets.astype(jnp.int32)
    ctx = contexts.astype(jnp.int32)
    neg_t = neg_samples.astype(jnp.int32).T
    wh2 = W_hidden.reshape(-1, _PD)
    wo2 = W_output.reshape(-1, _PD)
    pos_score, neg_score = _sc_scores(tgt, ctx, neg_t, wh2, wo2)
    return _tc_loss(pos_score.reshape(128, 128), neg_score.reshape(128, 128))[0, 0]


# speedup vs baseline: 1.1515x; 1.1515x over previous
import functools

import jax
import jax.numpy as jnp
from jax import lax
from jax.experimental import pallas as pl
from jax.experimental.pallas import tpu as pltpu
from jax.experimental.pallas import tpu_sc as plsc

_B = 16384
_D = 64
_K = 5
_NC = 2
_NS = 16
_NW = _NC * _NS
_BPW = _B // _NW
_CHUNK = 64
_NCHUNK = _BPW // _CHUNK
_LANES = 16
_LP = _D // _LANES
_PD = 128


def _sc_partials(targets, contexts, neg_t, w_hidden_p, w_output_p):
    mesh = plsc.VectorSubcoreMesh(
        core_axis_name="c", subcore_axis_name="s",
        num_cores=_NC, num_subcores=_NS)

    @functools.partial(
        pl.kernel,
        out_type=(
            jax.ShapeDtypeStruct((_B, _LANES), jnp.float32),
            jax.ShapeDtypeStruct((_B, _LANES), jnp.float32),
        ),
        mesh=mesh,
        scratch_types=[
            pltpu.VMEM((_CHUNK,), jnp.int32),
            pltpu.VMEM((_CHUNK,), jnp.int32),
            pltpu.VMEM((_K, _CHUNK), jnp.int32),
            pltpu.VMEM((_CHUNK, _PD), jnp.float32),
            pltpu.VMEM((_CHUNK, _PD), jnp.float32),
            pltpu.VMEM((_K, _CHUNK, _PD), jnp.float32),
            pltpu.VMEM((_CHUNK, _LANES), jnp.float32),
            pltpu.VMEM((_CHUNK, _LANES), jnp.float32),
            pltpu.SemaphoreType.DMA,
        ],
    )
    def sc_kernel(tgt_hbm, ctx_hbm, negt_hbm, wh_hbm, wo_hbm, pos_out, neg_out,
                  tgt_v, ctx_v, negi_v, h_v, o_v, n_v, pp_v, np_v, sem):
        wid = lax.axis_index("s") * _NC + lax.axis_index("c")

        def chunk_body(c, carry):
            base = wid * _BPW + c * _CHUNK
            pltpu.sync_copy(tgt_hbm.at[pl.ds(base, _CHUNK)], tgt_v)
            pltpu.sync_copy(ctx_hbm.at[pl.ds(base, _CHUNK)], ctx_v)
            for kk in range(_K):
                pltpu.sync_copy(negt_hbm.at[kk, pl.ds(base, _CHUNK)], negi_v.at[kk])
            copies = [
                pltpu.make_async_copy(wh_hbm.at[tgt_v], h_v, sem),
                pltpu.make_async_copy(wo_hbm.at[ctx_v], o_v, sem),
            ]
            for kk in range(_K):
                copies.append(
                    pltpu.make_async_copy(wo_hbm.at[negi_v.at[kk]], n_v.at[kk], sem))
            for cp in copies:
                cp.start()
            for cp in copies:
                cp.wait()

            def b_body(b, carry2):
                pp = jnp.zeros((_LANES,), jnp.float32)
                npart = jnp.zeros((_LANES,), jnp.float32)
                for j in range(_LP):
                    sl = pl.ds(j * _LANES, _LANES)
                    h = h_v[b, sl]
                    pp = pp + h * o_v[b, sl]
                    ns = n_v[0, b, sl]
                    for kk in range(1, _K):
                        ns = ns + n_v[kk, b, sl]
                    npart = npart + h * ns
                pp_v[b, :] = pp
                np_v[b, :] = npart
                return carry2

            lax.fori_loop(0, _CHUNK, b_body, 0)
            pltpu.sync_copy(pp_v, pos_out.at[pl.ds(base, _CHUNK)])
            pltpu.sync_copy(np_v, neg_out.at[pl.ds(base, _CHUNK)])
            return carry

        lax.fori_loop(0, _NCHUNK, chunk_body, 0)

    return sc_kernel(targets, contexts, neg_t, w_hidden_p, w_output_p)


def _log_sigmoid(x):
    return jnp.minimum(x, 0.0) - jnp.log1p(jnp.exp(-jnp.abs(x)))


def _tc_loss(pos_part, neg_part, *, interpret=False):
    def body(p_ref, n_ref, o_ref):
        score = jnp.sum(p_ref[...], axis=1, keepdims=True)
        negsum = jnp.sum(n_ref[...], axis=1, keepdims=True)
        total = jnp.sum(_log_sigmoid(score)) + jnp.sum(_log_sigmoid(-negsum))
        o_ref[0, 0] = -total * (1.0 / _B)

    return pl.pallas_call(
        body,
        out_shape=jax.ShapeDtypeStruct((1, 1), jnp.float32),
        out_specs=pl.BlockSpec(memory_space=pltpu.SMEM),
        interpret=interpret,
    )(pos_part, neg_part)


def kernel(targets, contexts, neg_samples, W_hidden, W_output):
    tgt = targets.astype(jnp.int32)
    ctx = contexts.astype(jnp.int32)
    neg_t = neg_samples.astype(jnp.int32).T
    wh_p = jnp.pad(W_hidden, ((0, 0), (0, _PD - _D)))
    wo_p = jnp.pad(W_output, ((0, 0), (0, _PD - _D)))
    pos_part, neg_part = _sc_partials(tgt, ctx, neg_t, wh_p, wo_p)
    return _tc_loss(pos_part, neg_part)[0, 0]
